# baseline (speedup 1.0000x reference)
import functools

import jax
import jax.numpy as jnp
from jax import lax
from jax.experimental import pallas as pl
from jax.experimental.pallas import tpu as pltpu
from jax.experimental.pallas import tpu_sc as plsc

_B = 16384
_DIMS = (8, 8, 8, 2, 4, 3, 64, 32, 16)
_PDIMS = (8, 8, 8, 8, 8, 8, 64, 32, 16)
_NT = len(_DIMS)
_TOTAL = sum(_PDIMS) + 1
_NC = 2
_NS = 16
_NW = _NC * _NS
_BPW = _B // _NW
_CHUNK = 128
_NCH = _BPW // _CHUNK

_OFFS = []
_o = 0
for _d in _PDIMS:
    _OFFS.append(_o)
    _o += _d


def _sc_gather(idx, *tables):
    mesh = plsc.VectorSubcoreMesh(core_axis_name="c", subcore_axis_name="s")
    out_type = tuple(jax.ShapeDtypeStruct((_B, d), jnp.float32) for d in _PDIMS)
    scratch = (
        [pltpu.VMEM((_NT, _NCH, _CHUNK), jnp.int32)]
        + [pltpu.VMEM((_BPW, d), jnp.float32) for d in _PDIMS]
        + [pltpu.SemaphoreType.DMA]
    )

    def body(*refs):
        idx_hbm = refs[0]
        tabs = refs[1:1 + _NT]
        outs = refs[1 + _NT:1 + 2 * _NT]
        idx_v = refs[1 + 2 * _NT]
        rows = refs[2 + 2 * _NT:2 + 3 * _NT]
        sem = refs[2 + 3 * _NT]

        wid = lax.axis_index("s") * _NC + lax.axis_index("c")
        base = wid * _BPW
        pltpu.sync_copy(idx_hbm.at[:, wid], idx_v)
        copies = []
        for t in range(_NT):
            for j in range(_NCH):
                copies.append(pltpu.async_copy(
                    tabs[t].at[idx_v.at[t, j]],
                    rows[t].at[pl.ds(j * _CHUNK, _CHUNK), :],
                    sem))
        for cp in copies:
            cp.wait()
        for t in range(_NT):
            pltpu.sync_copy(rows[t], outs[t].at[pl.ds(base, _BPW), :])

    fn = pl.kernel(body, out_type=out_type, mesh=mesh, scratch_types=scratch,
                   compiler_params=pltpu.CompilerParams(use_tc_tiling_on_sc=False))
    return fn(idx, *tables)


def _tc_mlp(gs, time_col, W1, b1, W2, b2, W3, b3):
    blk = 2048
    grid = (_B // blk,)

    def body(*refs):
        g = refs[:_NT]
        time_r, w1, b1r, w2, b2r, w3, b3r = refs[_NT:_NT + 7]
        out = refs[_NT + 7]
        w1v = w1[...]
        h = b1r[...] + time_r[...] * w1v[_TOTAL - 1:_TOTAL, :]
        for t in range(_NT):
            h = h + jnp.dot(g[t][...], w1v[_OFFS[t]:_OFFS[t] + _PDIMS[t], :],
                            preferred_element_type=jnp.float32)
        h = jnp.maximum(h, 0.0)
        h = jnp.maximum(jnp.dot(h, w2[...], preferred_element_type=jnp.float32)
                        + b2r[...], 0.0)
        o = jnp.dot(h, w3[...], preferred_element_type=jnp.float32) + b3r[...]
        out[...] = jax.nn.sigmoid(o)

    in_specs = (
        [pl.BlockSpec((blk, d), lambda i: (i, 0)) for d in _PDIMS]
        + [pl.BlockSpec((blk, 1), lambda i: (i, 0)),
           pl.BlockSpec((_TOTAL, 16), lambda i: (0, 0)),
           pl.BlockSpec((1, 16), lambda i: (0, 0)),
           pl.BlockSpec((16, 4), lambda i: (0, 0)),
           pl.BlockSpec((1, 4), lambda i: (0, 0)),
           pl.BlockSpec((4, 1), lambda i: (0, 0)),
           pl.BlockSpec((1, 1), lambda i: (0, 0))]
    )
    return pl.pallas_call(
        body,
        grid=grid,
        in_specs=in_specs,
        out_specs=pl.BlockSpec((blk, 1), lambda i: (i, 0)),
        out_shape=jax.ShapeDtypeStruct((_B, 1), jnp.float32),
    )(*gs, time_col, W1, b1, W2, b2, W3, b3)


def kernel(x, user_emb, product_emb, model_emb, gender_emb, age_emb,
           residence_emb, color_emb, size_emb, material_emb,
           W1, b1, W2, b2, W3, b3):
    idx = x[:, :_NT].astype(jnp.int32)
    idx_t = idx.T.reshape(_NT, _NW, _NCH, _CHUNK)
    time_col = x[:, _NT:_NT + 1]
    tables = (user_emb, product_emb, model_emb, gender_emb, age_emb,
              residence_emb, color_emb, size_emb, material_emb)
    tables = tuple(
        t if d == p else jnp.pad(t, ((0, 0), (0, p - d)))
        for t, d, p in zip(tables, _DIMS, _PDIMS))
    w1_rows = []
    o = 0
    for d, p in zip(_DIMS, _PDIMS):
        w1_rows.append(W1[o:o + d])
        if p > d:
            w1_rows.append(jnp.zeros((p - d, 16), W1.dtype))
        o += d
    w1_rows.append(W1[o:o + 1])
    W1p = jnp.concatenate(w1_rows, axis=0)
    gs = _sc_gather(idx_t, *tables)
    out = _tc_mlp(gs, time_col, W1p, b1.reshape(1, 16), W2, b2.reshape(1, 4),
                  W3, b3.reshape(1, 1))
    return out[:, 0]


# speedup vs baseline: 1.3915x; 1.3915x over previous
import jax
import jax.numpy as jnp
from jax import lax
from jax.experimental import pallas as pl
from jax.experimental.pallas import tpu as pltpu
from jax.experimental.pallas import tpu_sc as plsc

_B = 16384
_DIMS = (8, 8, 8, 2, 4, 3, 64, 32, 16)
_VOCABS = (1000000, 100000, 1000, 2, 10, 5, 64, 32, 16)
_NT = len(_DIMS)
_NSC = 3
_NC = 2
_NS = 16
_NW = _NC * _NS
_BPW = _B // _NW
_CHUNK = 128
_NCH = _BPW // _CHUNK
_L = 16

_OFFS = []
_o = 0
for _d in _DIMS:
    _OFFS.append(_o)
    _o += _d
_TIME_ROW = _o


def _sc_gather(x, *tables):
    mesh = plsc.VectorSubcoreMesh(core_axis_name="c", subcore_axis_name="s")
    out_type = tuple(jax.ShapeDtypeStruct((_B, 8), jnp.float32)
                     for _ in range(_NSC))
    scratch = (
        [pltpu.VMEM((_BPW, 10), jnp.float32),
         pltpu.VMEM((_NSC, _NCH, _CHUNK), jnp.int32)]
        + [pltpu.VMEM((_BPW, 8), jnp.float32) for _ in range(_NSC)]
        + [pltpu.SemaphoreType.DMA]
    )

    def body(x_hbm, *refs):
        tabs = refs[:_NSC]
        outs = refs[_NSC:2 * _NSC]
        xv = refs[2 * _NSC]
        idx_v = refs[2 * _NSC + 1]
        rows = refs[2 * _NSC + 2:2 * _NSC + 2 + _NSC]
        sem = refs[2 * _NSC + 2 + _NSC]

        wid = lax.axis_index("s") * _NC + lax.axis_index("c")
        base = wid * _BPW
        pltpu.sync_copy(x_hbm.at[pl.ds(base, _BPW), :], xv)
        lanes = lax.iota(jnp.int32, _L)
        for t in range(_NSC):
            col = jnp.full((_L,), t, jnp.int32)
            for c in range(_BPW // _L):
                vals = plsc.load_gather(xv, [c * _L + lanes, col])
                j, o = divmod(c * _L, _CHUNK)
                idx_v[t, j, pl.ds(o, _L)] = vals.astype(jnp.int32)
        copies = []
        for t in range(_NSC):
            for j in range(_NCH):
                copies.append(pltpu.async_copy(
                    tabs[t].at[idx_v.at[t, j]],
                    rows[t].at[pl.ds(j * _CHUNK, _CHUNK), :],
                    sem))
        for cp in copies:
            cp.wait()
        for t in range(_NSC):
            pltpu.sync_copy(rows[t], outs[t].at[pl.ds(base, _BPW), :])

    fn = pl.kernel(body, out_type=out_type, mesh=mesh, scratch_types=scratch,
                   compiler_params=pltpu.CompilerParams(
                       use_tc_tiling_on_sc=False, needs_layout_passes=False))
    return fn(x, *tables)


def _tc_mlp(x, gs, small_tables, W1, b1, W2, b2, W3, b3):
    blk = 2048
    grid = (_B // blk,)

    def body(*refs):
        x_ref = refs[0]
        g = refs[1:1 + _NSC]
        tt = refs[1 + _NSC:1 + _NSC + 6]
        w1, b1r, w2, b2r, w3, b3r = refs[1 + _NSC + 6:1 + _NSC + 12]
        out = refs[1 + _NSC + 12]

        xb = x_ref[...]
        w1v = w1[...]
        h = b1r[...] + xb[:, 9:10] * w1v[_TIME_ROW:_TIME_ROW + 1, :]
        for t in range(_NSC):
            h = h + jnp.dot(g[t][...], w1v[_OFFS[t]:_OFFS[t] + 8, :],
                            preferred_element_type=jnp.float32)
        for k in range(6):
            t = _NSC + k
            v = _VOCABS[t] + 1
            proj = jnp.dot(tt[k][...], w1v[_OFFS[t]:_OFFS[t] + _DIMS[t], :],
                           preferred_element_type=jnp.float32)
            ids = lax.broadcasted_iota(jnp.int32, (blk, v), 1)
            onehot = (xb[:, t:t + 1].astype(jnp.int32) == ids).astype(jnp.float32)
            h = h + jnp.dot(onehot, proj, preferred_element_type=jnp.float32)
        h = jnp.maximum(h, 0.0)
        h = jnp.maximum(jnp.dot(h, w2[...], preferred_element_type=jnp.float32)
                        + b2r[...], 0.0)
        o = jnp.dot(h, w3[...], preferred_element_type=jnp.float32) + b3r[...]
        out[...] = jax.nn.sigmoid(o)

    in_specs = (
        [pl.BlockSpec((blk, 10), lambda i: (i, 0))]
        + [pl.BlockSpec((blk, 8), lambda i: (i, 0)) for _ in range(_NSC)]
        + [pl.BlockSpec(t.shape, lambda i: (0, 0)) for t in small_tables]
        + [pl.BlockSpec((sum(_DIMS) + 1, 16), lambda i: (0, 0)),
           pl.BlockSpec((1, 16), lambda i: (0, 0)),
           pl.BlockSpec((16, 4), lambda i: (0, 0)),
           pl.BlockSpec((1, 4), lambda i: (0, 0)),
           pl.BlockSpec((4, 1), lambda i: (0, 0)),
           pl.BlockSpec((1, 1), lambda i: (0, 0))]
    )
    return pl.pallas_call(
        body,
        grid=grid,
        in_specs=in_specs,
        out_specs=pl.BlockSpec((blk, 1), lambda i: (i, 0)),
        out_shape=jax.ShapeDtypeStruct((_B, 1), jnp.float32),
    )(x, *gs, *small_tables, W1, b1, W2, b2, W3, b3)


def kernel(x, user_emb, product_emb, model_emb, gender_emb, age_emb,
           residence_emb, color_emb, size_emb, material_emb,
           W1, b1, W2, b2, W3, b3):
    gs = _sc_gather(x, user_emb, product_emb, model_emb)
    small_tables = (gender_emb, age_emb, residence_emb, color_emb, size_emb,
                    material_emb)
    out = _tc_mlp(x, gs, small_tables, W1, b1.reshape(1, 16), W2,
                  b2.reshape(1, 4), W3, b3.reshape(1, 1))
    return out[:, 0]


# speedup vs baseline: 3.0269x; 2.1752x over previous
import jax
import jax.numpy as jnp
from jax import lax
from jax.experimental import pallas as pl
from jax.experimental.pallas import tpu as pltpu
from jax.experimental.pallas import tpu_sc as plsc

_B = 16384
_DIMS = (8, 8, 8, 2, 4, 3, 64, 32, 16)
_VOCABS = (1000000, 100000, 1000, 2, 10, 5, 64, 32, 16)
_NSC = 3
_NC = 2
_NS = 16
_NW = _NC * _NS
_BPW = _B // _NW
_CHUNK = 128
_L = 16

_NTILES = tuple(-(-(_VOCABS[t] + 1) // 128) for t in range(_NSC))
_KT = 16


def _detile(ut, utail, pt, ptail, mt):
    mesh = plsc.VectorSubcoreMesh(core_axis_name="c", subcore_axis_name="s")
    out_type = tuple(jax.ShapeDtypeStruct((n * 8, 128), jnp.float32)
                     for n in _NTILES)

    def body(u_hbm, ut_hbm, p_hbm, pt_hbm, m_hbm, uo, po, mo,
             tiles_v, st_v, sem):
        wid = lax.axis_index("s") * _NC + lax.axis_index("c")
        lanes = lax.iota(jnp.int32, _L)
        dvec = lanes % 8
        kbase = lanes // 8

        def transpose_tiles(n):
            def tbody(k, _):
                for q in range(8):
                    for g in range(8):
                        lv = k * 128 + 16 * q + 2 * g + kbase
                        vals = plsc.load_gather(tiles_v, [dvec, lv])
                        st_v[k * 8 + q, pl.ds(g * _L, _L)] = vals
                return 0
            lax.fori_loop(0, n, tbody, 0)

        def chunk(src, dst, src_t0, dst_t0, n):
            pltpu.sync_copy(
                src.at[:, pl.ds(pl.multiple_of(src_t0 * 128, 128), n * 128)],
                tiles_v.at[:, pl.ds(0, n * 128)])
            transpose_tiles(n)
            pltpu.sync_copy(
                st_v.at[pl.ds(0, n * 8), :],
                dst.at[pl.ds(pl.multiple_of(dst_t0 * 8, 8), n * 8), :])

        def chunk_loop(src, dst, nchunks):
            def sbody(s, _):
                c = s * _NW + wid
                @pl.when(c < nchunks)
                def _():
                    chunk(src, dst, c * _KT, c * _KT, _KT)
                return 0
            lax.fori_loop(0, -(-nchunks // _NW), sbody, 0)

        chunk_loop(u_hbm, uo, 488)
        @pl.when(wid == 0)
        def _():
            chunk(ut_hbm, uo, 0, 488 * _KT, 5)
        chunk_loop(p_hbm, po, 48)
        @pl.when(wid == 1)
        def _():
            chunk(pt_hbm, po, 0, 48 * _KT, 14)
        @pl.when(wid == 2)
        def _():
            chunk(m_hbm, mo, 0, 0, 8)

    fn = pl.kernel(
        body, out_type=out_type, mesh=mesh,
        scratch_types=[pltpu.VMEM((8, _KT * 128), jnp.float32),
                       pltpu.VMEM((_KT * 8, 128), jnp.float32),
                       pltpu.SemaphoreType.DMA],
        compiler_params=pltpu.CompilerParams(
            use_tc_tiling_on_sc=True, needs_layout_passes=False))
    return fn(ut, utail, pt, ptail, mt)


def _sc_gather(xt, *tables):
    mesh = plsc.VectorSubcoreMesh(core_axis_name="c", subcore_axis_name="s")
    out_type = jax.ShapeDtypeStruct((8 * _NSC, _B), jnp.float32)
    scratch = (
        [pltpu.VMEM((_BPW,), jnp.float32),
         pltpu.VMEM((_NSC, _BPW // _CHUNK, _CHUNK), jnp.int32)]
        + [pltpu.VMEM((_BPW, 8), jnp.float32) for _ in range(_NSC)]
        + [pltpu.VMEM((8 * _NSC, _BPW), jnp.float32),
           pltpu.SemaphoreType.DMA]
    )

    def body(xt_hbm, *refs):
        tabs = refs[:_NSC]
        out = refs[_NSC]
        xcol = refs[_NSC + 1]
        idx_v = refs[_NSC + 2]
        rows = refs[_NSC + 3:_NSC + 3 + _NSC]
        st = refs[_NSC + 3 + _NSC]
        sem = refs[_NSC + 4 + _NSC]

        wid = lax.axis_index("s") * _NC + lax.axis_index("c")
        base = wid * _BPW
        lanes = lax.iota(jnp.int32, _L)
        for t in range(_NSC):
            pltpu.sync_copy(xt_hbm.at[t, pl.ds(base, _BPW)], xcol)
            for c in range(_BPW // _L):
                vals = xcol[pl.ds(c * _L, _L)].astype(jnp.int32)
                j, o = divmod(c * _L, _CHUNK)
                idx_v[t, j, pl.ds(o, _L)] = vals
        copies = []
        for t in range(_NSC):
            for j in range(_BPW // _CHUNK):
                copies.append(pltpu.async_copy(
                    tabs[t].at[idx_v.at[t, j]],
                    rows[t].at[pl.ds(j * _CHUNK, _CHUNK), :], sem))
        for cp in copies:
            cp.wait()
        for t in range(_NSC):
            for d in range(8):
                dv = jnp.full((_L,), d, jnp.int32)
                for g in range(_BPW // _L):
                    vals = plsc.load_gather(rows[t], [g * _L + lanes, dv])
                    st[t * 8 + d, pl.ds(g * _L, _L)] = vals
        pltpu.sync_copy(st, out.at[:, pl.ds(base, _BPW)])

    fn = pl.kernel(
        body, out_type=out_type, mesh=mesh, scratch_types=scratch,
        compiler_params=pltpu.CompilerParams(
            use_tc_tiling_on_sc=False, needs_layout_passes=False))
    return fn(xt, *tables)


def _tc_mlp(xt, gt, small_t, W1t, b1, W2t, b2, W3t, b3):
    blk = 2048
    grid = (_B // blk,)
    offs = []
    o = 0
    for d in _DIMS:
        offs.append(o)
        o += d
    time_row = o

    def body(*refs):
        x_ref, g_ref = refs[0], refs[1]
        tt = refs[2:8]
        w1t, b1r, w2t, b2r, w3t, b3r = refs[8:14]
        out = refs[14]

        xb = x_ref[...]
        gb = g_ref[...]
        w1v = w1t[...]
        h = b1r[...] + w1v[:, time_row:time_row + 1] * xb[9:10, :]
        h = h + jnp.dot(w1v[:, 0:24], gb, preferred_element_type=jnp.float32)
        for k in range(6):
            t = _NSC + k
            v = _VOCABS[t] + 1
            projT = jnp.dot(w1v[:, offs[t]:offs[t] + _DIMS[t]], tt[k][...],
                            preferred_element_type=jnp.float32)
            ids = lax.broadcasted_iota(jnp.int32, (v, blk), 0)
            onehotT = (ids == xb[t:t + 1, :].astype(jnp.int32)).astype(jnp.float32)
            h = h + jnp.dot(projT, onehotT, preferred_element_type=jnp.float32)
        h = jnp.maximum(h, 0.0)
        h = jnp.maximum(jnp.dot(w2t[...], h, preferred_element_type=jnp.float32)
                        + b2r[...], 0.0)
        o1 = jnp.dot(w3t[...], h, preferred_element_type=jnp.float32) + b3r[...]
        out[...] = jnp.broadcast_to(jax.nn.sigmoid(o1), (8, blk))

    in_specs = (
        [pl.BlockSpec((10, blk), lambda i: (0, i)),
         pl.BlockSpec((8 * _NSC, blk), lambda i: (0, i))]
        + [pl.BlockSpec(t.shape, lambda i: (0, 0)) for t in small_t]
        + [pl.BlockSpec((16, 146), lambda i: (0, 0)),
           pl.BlockSpec((16, 1), lambda i: (0, 0)),
           pl.BlockSpec((4, 16), lambda i: (0, 0)),
           pl.BlockSpec((4, 1), lambda i: (0, 0)),
           pl.BlockSpec((1, 4), lambda i: (0, 0)),
           pl.BlockSpec((1, 1), lambda i: (0, 0))]
    )
    return pl.pallas_call(
        body,
        grid=grid,
        in_specs=in_specs,
        out_specs=pl.BlockSpec((8, blk), lambda i: (0, i)),
        out_shape=jax.ShapeDtypeStruct((8, _B), jnp.float32),
    )(xt, gt, *small_t, W1t, b1, W2t, b2, W3t, b3)


def kernel(x, user_emb, product_emb, model_emb, gender_emb, age_emb,
           residence_emb, color_emb, size_emb, material_emb,
           W1, b1, W2, b2, W3, b3):
    xt = x.T
    utail = jnp.pad(user_emb[488 * _KT * 128:], ((0, 5 * 128 - 577), (0, 0)))
    ptail = jnp.pad(product_emb[48 * _KT * 128:], ((0, 14 * 128 - 1697), (0, 0)))
    mpad = jnp.pad(model_emb, ((0, 8 * 128 - 1001), (0, 0)))
    uo, po, mo = _detile(user_emb.T, utail.T, product_emb.T, ptail.T, mpad.T)
    big = tuple(o.reshape(n * 8 * 16, 8) for o, n in zip((uo, po, mo), _NTILES))
    gt = _sc_gather(xt, *big)
    small_t = (gender_emb.T, age_emb.T, residence_emb.T, color_emb.T,
               size_emb.T, material_emb.T)
    out = _tc_mlp(xt, gt, small_t, W1.T, b1.reshape(16, 1), W2.T,
                  b2.reshape(4, 1), W3.T, b3.reshape(1, 1))
    return out[0]


# speedup vs baseline: 3.4744x; 1.1479x over previous
import jax
import jax.numpy as jnp
from jax import lax
from jax.experimental import pallas as pl
from jax.experimental.pallas import tpu as pltpu
from jax.experimental.pallas import tpu_sc as plsc

_B = 16384
_DIMS = (8, 8, 8, 2, 4, 3, 64, 32, 16)
_VOCABS = (1000000, 100000, 1000, 2, 10, 5, 64, 32, 16)
_NSC = 3
_NC = 2
_NS = 16
_NW = _NC * _NS
_BPW = _B // _NW
_CHUNK = 128
_L = 16

_NTILES = tuple(-(-(_VOCABS[t] + 1) // 128) for t in range(_NSC))
_KT = 16


def _detile(ut, utail, pt, ptail, mt):
    mesh = plsc.VectorSubcoreMesh(core_axis_name="c", subcore_axis_name="s")
    out_type = tuple(jax.ShapeDtypeStruct((n * 8, 128), jnp.float32)
                     for n in _NTILES)

    def body(u_hbm, ut_hbm, p_hbm, pt_hbm, m_hbm, uo, po, mo,
             t0_v, t1_v, s0_v, s1_v, semi0, semi1, semo0, semo1):
        wid = lax.axis_index("s") * _NC + lax.axis_index("c")
        lanes = lax.iota(jnp.int32, _L)
        dvec = lanes % 8
        kbase = lanes // 8
        tiles = (t0_v, t1_v)
        sts = (s0_v, s1_v)
        semis = (semi0, semi1)
        semos = (semo0, semo1)

        def transpose_tiles(tiles_v, st_v, n):
            def tbody(k, _):
                for q in range(8):
                    for g in range(8):
                        lv = k * 128 + 16 * q + 2 * g + kbase
                        vals = plsc.load_gather(tiles_v, [dvec, lv])
                        st_v[k * 8 + q, pl.ds(g * _L, _L)] = vals
                return 0
            lax.fori_loop(0, n, tbody, 0)

        def chunk(src, dst, src_t0, dst_t0, n):
            pltpu.sync_copy(
                src.at[:, pl.ds(pl.multiple_of(src_t0 * 128, 128), n * 128)],
                t0_v.at[:, pl.ds(0, n * 128)])
            transpose_tiles(t0_v, s0_v, n)
            pltpu.sync_copy(
                s0_v.at[pl.ds(0, n * 8), :],
                dst.at[pl.ds(pl.multiple_of(dst_t0 * 8, 8), n * 8), :])

        def chunk_loop(src, dst, nchunks, spw):
            def mk_in(s, b):
                c = pl.multiple_of((s * _NW + wid) * _KT * 128, 128)
                return pltpu.make_async_copy(
                    src.at[:, pl.ds(c, _KT * 128)], tiles[b], semis[b])

            def mk_out(s, b):
                c = pl.multiple_of((s * _NW + wid) * _KT * 8, 8)
                return pltpu.make_async_copy(
                    sts[b], dst.at[pl.ds(c, _KT * 8), :], semos[b])

            for b in range(2):
                @pl.when(b * _NW + wid < nchunks)
                def _():
                    mk_in(b, b).start()

            def sbody(s2, _):
                for b in range(2):
                    s = 2 * s2 + b
                    c = s * _NW + wid
                    @pl.when(c < nchunks)
                    def _():
                        mk_in(s, b).wait()
                        @pl.when(s >= 2)
                        def _():
                            mk_out(s - 2, b).wait()
                        transpose_tiles(tiles[b], sts[b], _KT)
                        mk_out(s, b).start()
                        @pl.when((s + 2) * _NW + wid < nchunks)
                        def _():
                            mk_in(s + 2, b).start()
                return 0
            lax.fori_loop(0, spw // 2, sbody, 0)
            for s in range(max(0, spw - 3), spw):
                @pl.when((s * _NW + wid < nchunks)
                         & ((s + 2) * _NW + wid >= nchunks))
                def _():
                    mk_out(s, s % 2).wait()

        chunk_loop(u_hbm, uo, 488, 16)
        @pl.when(wid == 0)
        def _():
            chunk(ut_hbm, uo, 0, 488 * _KT, 5)
        chunk_loop(p_hbm, po, 48, 2)
        @pl.when(wid == 1)
        def _():
            chunk(pt_hbm, po, 0, 48 * _KT, 14)
        @pl.when(wid == 2)
        def _():
            chunk(m_hbm, mo, 0, 0, 8)

    fn = pl.kernel(
        body, out_type=out_type, mesh=mesh,
        scratch_types=[pltpu.VMEM((8, _KT * 128), jnp.float32),
                       pltpu.VMEM((8, _KT * 128), jnp.float32),
                       pltpu.VMEM((_KT * 8, 128), jnp.float32),
                       pltpu.VMEM((_KT * 8, 128), jnp.float32),
                       pltpu.SemaphoreType.DMA,
                       pltpu.SemaphoreType.DMA,
                       pltpu.SemaphoreType.DMA,
                       pltpu.SemaphoreType.DMA],
        compiler_params=pltpu.CompilerParams(
            use_tc_tiling_on_sc=True, needs_layout_passes=False))
    return fn(ut, utail, pt, ptail, mt)


def _sc_gather(xt, *tables):
    mesh = plsc.VectorSubcoreMesh(core_axis_name="c", subcore_axis_name="s")
    out_type = jax.ShapeDtypeStruct((8 * _NSC, _B), jnp.float32)
    scratch = (
        [pltpu.VMEM((_BPW,), jnp.float32),
         pltpu.VMEM((_NSC, _BPW // _CHUNK, _CHUNK), jnp.int32)]
        + [pltpu.VMEM((_BPW, 8), jnp.float32) for _ in range(_NSC)]
        + [pltpu.VMEM((8 * _NSC, _BPW), jnp.float32),
           pltpu.SemaphoreType.DMA]
    )

    def body(xt_hbm, *refs):
        tabs = refs[:_NSC]
        out = refs[_NSC]
        xcol = refs[_NSC + 1]
        idx_v = refs[_NSC + 2]
        rows = refs[_NSC + 3:_NSC + 3 + _NSC]
        st = refs[_NSC + 3 + _NSC]
        sem = refs[_NSC + 4 + _NSC]

        wid = lax.axis_index("s") * _NC + lax.axis_index("c")
        base = wid * _BPW
        lanes = lax.iota(jnp.int32, _L)
        for t in range(_NSC):
            pltpu.sync_copy(xt_hbm.at[t, pl.ds(base, _BPW)], xcol)
            for c in range(_BPW // _L):
                vals = xcol[pl.ds(c * _L, _L)].astype(jnp.int32)
                j, o = divmod(c * _L, _CHUNK)
                idx_v[t, j, pl.ds(o, _L)] = vals
        copies = []
        for t in range(_NSC):
            for j in range(_BPW // _CHUNK):
                copies.append(pltpu.async_copy(
                    tabs[t].at[idx_v.at[t, j]],
                    rows[t].at[pl.ds(j * _CHUNK, _CHUNK), :], sem))
        for cp in copies:
            cp.wait()
        for t in range(_NSC):
            for d in range(8):
                dv = jnp.full((_L,), d, jnp.int32)
                for g in range(_BPW // _L):
                    vals = plsc.load_gather(rows[t], [g * _L + lanes, dv])
                    st[t * 8 + d, pl.ds(g * _L, _L)] = vals
        pltpu.sync_copy(st, out.at[:, pl.ds(base, _BPW)])

    fn = pl.kernel(
        body, out_type=out_type, mesh=mesh, scratch_types=scratch,
        compiler_params=pltpu.CompilerParams(
            use_tc_tiling_on_sc=False, needs_layout_passes=False))
    return fn(xt, *tables)


def _tc_mlp(xt, gt, small_t, W1t, b1, W2t, b2, W3t, b3):
    blk = 2048
    grid = (_B // blk,)
    offs = []
    o = 0
    for d in _DIMS:
        offs.append(o)
        o += d
    time_row = o

    def body(*refs):
        x_ref, g_ref = refs[0], refs[1]
        tt = refs[2:8]
        w1t, b1r, w2t, b2r, w3t, b3r = refs[8:14]
        out = refs[14]

        xb = x_ref[...]
        gb = g_ref[...]
        w1v = w1t[...]
        h = b1r[...] + w1v[:, time_row:time_row + 1] * xb[9:10, :]
        h = h + jnp.dot(w1v[:, 0:24], gb, preferred_element_type=jnp.float32)
        for k in range(6):
            t = _NSC + k
            v = _VOCABS[t] + 1
            projT = jnp.dot(w1v[:, offs[t]:offs[t] + _DIMS[t]], tt[k][...],
                            preferred_element_type=jnp.float32)
            ids = lax.broadcasted_iota(jnp.int32, (v, blk), 0)
            onehotT = (ids == xb[t:t + 1, :].astype(jnp.int32)).astype(jnp.float32)
            h = h + jnp.dot(projT, onehotT, preferred_element_type=jnp.float32)
        h = jnp.maximum(h, 0.0)
        h = jnp.maximum(jnp.dot(w2t[...], h, preferred_element_type=jnp.float32)
                        + b2r[...], 0.0)
        o1 = jnp.dot(w3t[...], h, preferred_element_type=jnp.float32) + b3r[...]
        out[...] = jnp.broadcast_to(jax.nn.sigmoid(o1), (8, blk))

    in_specs = (
        [pl.BlockSpec((10, blk), lambda i: (0, i)),
         pl.BlockSpec((8 * _NSC, blk), lambda i: (0, i))]
        + [pl.BlockSpec(t.shape, lambda i: (0, 0)) for t in small_t]
        + [pl.BlockSpec((16, 146), lambda i: (0, 0)),
           pl.BlockSpec((16, 1), lambda i: (0, 0)),
           pl.BlockSpec((4, 16), lambda i: (0, 0)),
           pl.BlockSpec((4, 1), lambda i: (0, 0)),
           pl.BlockSpec((1, 4), lambda i: (0, 0)),
           pl.BlockSpec((1, 1), lambda i: (0, 0))]
    )
    return pl.pallas_call(
        body,
        grid=grid,
        in_specs=in_specs,
        out_specs=pl.BlockSpec((8, blk), lambda i: (0, i)),
        out_shape=jax.ShapeDtypeStruct((8, _B), jnp.float32),
    )(xt, gt, *small_t, W1t, b1, W2t, b2, W3t, b3)


def kernel(x, user_emb, product_emb, model_emb, gender_emb, age_emb,
           residence_emb, color_emb, size_emb, material_emb,
           W1, b1, W2, b2, W3, b3):
    xt = x.T
    utail = jnp.pad(user_emb[488 * _KT * 128:], ((0, 5 * 128 - 577), (0, 0)))
    ptail = jnp.pad(product_emb[48 * _KT * 128:], ((0, 14 * 128 - 1697), (0, 0)))
    mpad = jnp.pad(model_emb, ((0, 8 * 128 - 1001), (0, 0)))
    uo, po, mo = _detile(user_emb.T, utail.T, product_emb.T, ptail.T, mpad.T)
    big = tuple(o.reshape(n * 8 * 16, 8) for o, n in zip((uo, po, mo), _NTILES))
    gt = _sc_gather(xt, *big)
    small_t = (gender_emb.T, age_emb.T, residence_emb.T, color_emb.T,
               size_emb.T, material_emb.T)
    out = _tc_mlp(xt, gt, small_t, W1.T, b1.reshape(16, 1), W2.T,
                  b2.reshape(4, 1), W3.T, b3.reshape(1, 1))
    return out[0]


# speedup vs baseline: 5.6353x; 1.6219x over previous
import jax
import jax.numpy as jnp
from jax import lax
from jax.experimental import pallas as pl
from jax.experimental.pallas import tpu as pltpu
from jax.experimental.pallas import tpu_sc as plsc

_B = 16384
_DIMS = (8, 8, 8, 2, 4, 3, 64, 32, 16)
_VOCABS = (1000000, 100000, 1000, 2, 10, 5, 64, 32, 16)
_NSC = 3
_NC = 2
_NS = 16
_NW = _NC * _NS
_BPW = _B // _NW
_CHUNK = 128
_L = 16

_NTILES = tuple(-(-(_VOCABS[t] + 1) // 128) for t in range(_NSC))
_KT = 16


def _detile(ut, utail, pt, ptail, mt):
    mesh = plsc.VectorSubcoreMesh(core_axis_name="c", subcore_axis_name="s")
    out_type = tuple(jax.ShapeDtypeStruct((n * 8, 128), jnp.float32)
                     for n in _NTILES)

    def body(u_hbm, ut_hbm, p_hbm, pt_hbm, m_hbm, uo, po, mo,
             t0_v, t1_v, s0_v, s1_v, semi0, semi1, semo0, semo1):
        wid = lax.axis_index("s") * _NC + lax.axis_index("c")
        lanes = lax.iota(jnp.int32, _L)
        dvec = lanes % 8
        kbase = lanes // 8
        tiles = (t0_v, t1_v)
        sts = (s0_v, s1_v)
        semis = (semi0, semi1)
        semos = (semo0, semo1)

        colvecs = [lanes * 8 + d for d in range(8)]

        def transpose_tiles(tiles_v, st_v, n):
            def tbody(k, _):
                for q in range(8):
                    row = jnp.full((_L,), k * 8 + q, jnp.int32)
                    for d in range(8):
                        vals = tiles_v[d, pl.ds(k * 128 + 16 * q, _L)]
                        plsc.store_scatter(st_v, [row, colvecs[d]], vals)
                return 0
            lax.fori_loop(0, n, tbody, 0)

        def chunk(src, dst, src_t0, dst_t0, n):
            pltpu.sync_copy(
                src.at[:, pl.ds(pl.multiple_of(src_t0 * 128, 128), n * 128)],
                t0_v.at[:, pl.ds(0, n * 128)])
            transpose_tiles(t0_v, s0_v, n)
            pltpu.sync_copy(
                s0_v.at[pl.ds(0, n * 8), :],
                dst.at[pl.ds(pl.multiple_of(dst_t0 * 8, 8), n * 8), :])

        def chunk_loop(src, dst, nchunks, spw):
            def mk_in(s, b):
                c = pl.multiple_of((s * _NW + wid) * _KT * 128, 128)
                return pltpu.make_async_copy(
                    src.at[:, pl.ds(c, _KT * 128)], tiles[b], semis[b])

            def mk_out(s, b):
                c = pl.multiple_of((s * _NW + wid) * _KT * 8, 8)
                return pltpu.make_async_copy(
                    sts[b], dst.at[pl.ds(c, _KT * 8), :], semos[b])

            for b in range(2):
                @pl.when(b * _NW + wid < nchunks)
                def _():
                    mk_in(b, b).start()

            def sbody(s2, _):
                for b in range(2):
                    s = 2 * s2 + b
                    c = s * _NW + wid
                    @pl.when(c < nchunks)
                    def _():
                        mk_in(s, b).wait()
                        @pl.when(s >= 2)
                        def _():
                            mk_out(s - 2, b).wait()
                        transpose_tiles(tiles[b], sts[b], _KT)
                        mk_out(s, b).start()
                        @pl.when((s + 2) * _NW + wid < nchunks)
                        def _():
                            mk_in(s + 2, b).start()
                return 0
            lax.fori_loop(0, spw // 2, sbody, 0)
            for s in range(max(0, spw - 3), spw):
                @pl.when((s * _NW + wid < nchunks)
                         & ((s + 2) * _NW + wid >= nchunks))
                def _():
                    mk_out(s, s % 2).wait()

        chunk_loop(u_hbm, uo, 488, 16)
        @pl.when(wid == 0)
        def _():
            chunk(ut_hbm, uo, 0, 488 * _KT, 5)
        chunk_loop(p_hbm, po, 48, 2)
        @pl.when(wid == 1)
        def _():
            chunk(pt_hbm, po, 0, 48 * _KT, 14)
        @pl.when(wid == 2)
        def _():
            chunk(m_hbm, mo, 0, 0, 8)

    fn = pl.kernel(
        body, out_type=out_type, mesh=mesh,
        scratch_types=[pltpu.VMEM((8, _KT * 128), jnp.float32),
                       pltpu.VMEM((8, _KT * 128), jnp.float32),
                       pltpu.VMEM((_KT * 8, 128), jnp.float32),
                       pltpu.VMEM((_KT * 8, 128), jnp.float32),
                       pltpu.SemaphoreType.DMA,
                       pltpu.SemaphoreType.DMA,
                       pltpu.SemaphoreType.DMA,
                       pltpu.SemaphoreType.DMA],
        compiler_params=pltpu.CompilerParams(
            use_tc_tiling_on_sc=True, needs_layout_passes=False))
    return fn(ut, utail, pt, ptail, mt)


def _sc_gather(xt, *tables):
    mesh = plsc.VectorSubcoreMesh(core_axis_name="c", subcore_axis_name="s")
    out_type = jax.ShapeDtypeStruct((8 * _NSC, _B), jnp.float32)
    scratch = (
        [pltpu.VMEM((_BPW,), jnp.float32),
         pltpu.VMEM((_NSC, _BPW // _CHUNK, _CHUNK), jnp.int32)]
        + [pltpu.VMEM((_BPW, 8), jnp.float32) for _ in range(_NSC)]
        + [pltpu.VMEM((8 * _NSC, _BPW), jnp.float32),
           pltpu.SemaphoreType.DMA]
    )

    def body(xt_hbm, *refs):
        tabs = refs[:_NSC]
        out = refs[_NSC]
        xcol = refs[_NSC + 1]
        idx_v = refs[_NSC + 2]
        rows = refs[_NSC + 3:_NSC + 3 + _NSC]
        st = refs[_NSC + 3 + _NSC]
        sem = refs[_NSC + 4 + _NSC]

        wid = lax.axis_index("s") * _NC + lax.axis_index("c")
        base = wid * _BPW
        lanes = lax.iota(jnp.int32, _L)
        for t in range(_NSC):
            pltpu.sync_copy(xt_hbm.at[t, pl.ds(base, _BPW)], xcol)
            for c in range(_BPW // _L):
                vals = xcol[pl.ds(c * _L, _L)].astype(jnp.int32)
                j, o = divmod(c * _L, _CHUNK)
                idx_v[t, j, pl.ds(o, _L)] = vals
        copies = []
        for t in range(_NSC):
            for j in range(_BPW // _CHUNK):
                copies.append(pltpu.async_copy(
                    tabs[t].at[idx_v.at[t, j]],
                    rows[t].at[pl.ds(j * _CHUNK, _CHUNK), :], sem))
        for cp in copies:
            cp.wait()
        for t in range(_NSC):
            for d in range(8):
                dv = jnp.full((_L,), d, jnp.int32)
                for g in range(_BPW // _L):
                    vals = plsc.load_gather(rows[t], [g * _L + lanes, dv])
                    st[t * 8 + d, pl.ds(g * _L, _L)] = vals
        pltpu.sync_copy(st, out.at[:, pl.ds(base, _BPW)])

    fn = pl.kernel(
        body, out_type=out_type, mesh=mesh, scratch_types=scratch,
        compiler_params=pltpu.CompilerParams(
            use_tc_tiling_on_sc=False, needs_layout_passes=False))
    return fn(xt, *tables)


def _tc_mlp(xt, gt, small_t, W1t, b1, W2t, b2, W3t, b3):
    blk = 2048
    grid = (_B // blk,)
    offs = []
    o = 0
    for d in _DIMS:
        offs.append(o)
        o += d
    time_row = o

    def body(*refs):
        x_ref, g_ref = refs[0], refs[1]
        tt = refs[2:8]
        w1t, b1r, w2t, b2r, w3t, b3r = refs[8:14]
        out = refs[14]

        xb = x_ref[...]
        gb = g_ref[...]
        w1v = w1t[...]
        h = b1r[...] + w1v[:, time_row:time_row + 1] * xb[9:10, :]
        h = h + jnp.dot(w1v[:, 0:24], gb, preferred_element_type=jnp.float32)
        for k in range(6):
            t = _NSC + k
            v = _VOCABS[t] + 1
            projT = jnp.dot(w1v[:, offs[t]:offs[t] + _DIMS[t]], tt[k][...],
                            preferred_element_type=jnp.float32)
            ids = lax.broadcasted_iota(jnp.int32, (v, blk), 0)
            onehotT = (ids == xb[t:t + 1, :].astype(jnp.int32)).astype(jnp.float32)
            h = h + jnp.dot(projT, onehotT, preferred_element_type=jnp.float32)
        h = jnp.maximum(h, 0.0)
        h = jnp.maximum(jnp.dot(w2t[...], h, preferred_element_type=jnp.float32)
                        + b2r[...], 0.0)
        o1 = jnp.dot(w3t[...], h, preferred_element_type=jnp.float32) + b3r[...]
        out[...] = jnp.broadcast_to(jax.nn.sigmoid(o1), (8, blk))

    in_specs = (
        [pl.BlockSpec((10, blk), lambda i: (0, i)),
         pl.BlockSpec((8 * _NSC, blk), lambda i: (0, i))]
        + [pl.BlockSpec(t.shape, lambda i: (0, 0)) for t in small_t]
        + [pl.BlockSpec((16, 146), lambda i: (0, 0)),
           pl.BlockSpec((16, 1), lambda i: (0, 0)),
           pl.BlockSpec((4, 16), lambda i: (0, 0)),
           pl.BlockSpec((4, 1), lambda i: (0, 0)),
           pl.BlockSpec((1, 4), lambda i: (0, 0)),
           pl.BlockSpec((1, 1), lambda i: (0, 0))]
    )
    return pl.pallas_call(
        body,
        grid=grid,
        in_specs=in_specs,
        out_specs=pl.BlockSpec((8, blk), lambda i: (0, i)),
        out_shape=jax.ShapeDtypeStruct((8, _B), jnp.float32),
    )(xt, gt, *small_t, W1t, b1, W2t, b2, W3t, b3)


def kernel(x, user_emb, product_emb, model_emb, gender_emb, age_emb,
           residence_emb, color_emb, size_emb, material_emb,
           W1, b1, W2, b2, W3, b3):
    xt = x.T
    utail = jnp.pad(user_emb[488 * _KT * 128:], ((0, 5 * 128 - 577), (0, 0)))
    ptail = jnp.pad(product_emb[48 * _KT * 128:], ((0, 14 * 128 - 1697), (0, 0)))
    mpad = jnp.pad(model_emb, ((0, 8 * 128 - 1001), (0, 0)))
    uo, po, mo = _detile(user_emb.T, utail.T, product_emb.T, ptail.T, mpad.T)
    big = tuple(o.reshape(n * 8 * 16, 8) for o, n in zip((uo, po, mo), _NTILES))
    gt = _sc_gather(xt, *big)
    small_t = (gender_emb.T, age_emb.T, residence_emb.T, color_emb.T,
               size_emb.T, material_emb.T)
    out = _tc_mlp(xt, gt, small_t, W1.T, b1.reshape(16, 1), W2.T,
                  b2.reshape(4, 1), W3.T, b3.reshape(1, 1))
    return out[0]


# speedup vs baseline: 5.8690x; 1.0415x over previous
import jax
import jax.numpy as jnp
from jax import lax
from jax.experimental import pallas as pl
from jax.experimental.pallas import tpu as pltpu
from jax.experimental.pallas import tpu_sc as plsc

_B = 16384
_DIMS = (8, 8, 8, 2, 4, 3, 64, 32, 16)
_VOCABS = (1000000, 100000, 1000, 2, 10, 5, 64, 32, 16)
_NSC = 3
_NC = 2
_NS = 16
_NW = _NC * _NS
_BPW = _B // _NW
_CHUNK = 128
_L = 16

_NTILES = tuple(-(-(_VOCABS[t] + 1) // 128) for t in range(_NSC))
_KT = 16


def _detile(ut, utail, pt, ptail, mt):
    mesh = plsc.VectorSubcoreMesh(core_axis_name="c", subcore_axis_name="s")
    out_type = tuple(jax.ShapeDtypeStruct((n * 8, 128), jnp.float32)
                     for n in _NTILES)

    def body(u_hbm, ut_hbm, p_hbm, pt_hbm, m_hbm, uo, po, mo,
             t0_v, t1_v, s0_v, s1_v, semi0, semi1, semo0, semo1):
        wid = lax.axis_index("s") * _NC + lax.axis_index("c")
        lanes = lax.iota(jnp.int32, _L)
        dvec = lanes % 8
        kbase = lanes // 8
        tiles = (t0_v, t1_v)
        sts = (s0_v, s1_v)
        semis = (semi0, semi1)
        semos = (semo0, semo1)

        colvecs = [lanes * 8 + d for d in range(8)]

        def transpose_tiles(tiles_v, st_v, n):
            def tbody(k, _):
                for q in range(8):
                    row = jnp.full((_L,), k * 8 + q, jnp.int32)
                    for d in range(8):
                        vals = tiles_v[d, pl.ds(k * 128 + 16 * q, _L)]
                        plsc.store_scatter(st_v, [row, colvecs[d]], vals)
                return 0
            lax.fori_loop(0, n, tbody, 0)

        def chunk(src, dst, src_t0, dst_t0, n):
            pltpu.sync_copy(
                src.at[:, pl.ds(pl.multiple_of(src_t0 * 128, 128), n * 128)],
                t0_v.at[:, pl.ds(0, n * 128)])
            transpose_tiles(t0_v, s0_v, n)
            pltpu.sync_copy(
                s0_v.at[pl.ds(0, n * 8), :],
                dst.at[pl.ds(pl.multiple_of(dst_t0 * 8, 8), n * 8), :])

        def chunk_loop(src, dst, nchunks, spw):
            def mk_in(s, b):
                c = pl.multiple_of((s * _NW + wid) * _KT * 128, 128)
                return pltpu.make_async_copy(
                    src.at[:, pl.ds(c, _KT * 128)], tiles[b], semis[b])

            def mk_out(s, b):
                c = pl.multiple_of((s * _NW + wid) * _KT * 8, 8)
                return pltpu.make_async_copy(
                    sts[b], dst.at[pl.ds(c, _KT * 8), :], semos[b])

            for b in range(2):
                @pl.when(b * _NW + wid < nchunks)
                def _():
                    mk_in(b, b).start()

            def sbody(s2, _):
                for b in range(2):
                    s = 2 * s2 + b
                    c = s * _NW + wid
                    @pl.when(c < nchunks)
                    def _():
                        mk_in(s, b).wait()
                        @pl.when(s >= 2)
                        def _():
                            mk_out(s - 2, b).wait()
                        transpose_tiles(tiles[b], sts[b], _KT)
                        mk_out(s, b).start()
                        @pl.when((s + 2) * _NW + wid < nchunks)
                        def _():
                            mk_in(s + 2, b).start()
                return 0
            lax.fori_loop(0, spw // 2, sbody, 0)
            for s in range(max(0, spw - 3), spw):
                @pl.when((s * _NW + wid < nchunks)
                         & ((s + 2) * _NW + wid >= nchunks))
                def _():
                    mk_out(s, s % 2).wait()

        chunk_loop(u_hbm, uo, 488, 16)
        @pl.when(wid == 0)
        def _():
            chunk(ut_hbm, uo, 0, 488 * _KT, 5)
        chunk_loop(p_hbm, po, 48, 2)
        @pl.when(wid == 1)
        def _():
            chunk(pt_hbm, po, 0, 48 * _KT, 14)
        @pl.when(wid == 2)
        def _():
            chunk(m_hbm, mo, 0, 0, 8)

    fn = pl.kernel(
        body, out_type=out_type, mesh=mesh,
        scratch_types=[pltpu.VMEM((8, _KT * 128), jnp.float32),
                       pltpu.VMEM((8, _KT * 128), jnp.float32),
                       pltpu.VMEM((_KT * 8, 128), jnp.float32),
                       pltpu.VMEM((_KT * 8, 128), jnp.float32),
                       pltpu.SemaphoreType.DMA,
                       pltpu.SemaphoreType.DMA,
                       pltpu.SemaphoreType.DMA,
                       pltpu.SemaphoreType.DMA],
        compiler_params=pltpu.CompilerParams(
            use_tc_tiling_on_sc=True, needs_layout_passes=False))
    return fn(ut, utail, pt, ptail, mt)


def _sc_gather(xt, *tables):
    mesh = plsc.VectorSubcoreMesh(core_axis_name="c", subcore_axis_name="s")
    out_type = jax.ShapeDtypeStruct((8 * _NSC, _B), jnp.float32)
    scratch = (
        [pltpu.VMEM((_BPW,), jnp.float32),
         pltpu.VMEM((_NSC, _BPW // _CHUNK, _CHUNK), jnp.int32)]
        + [pltpu.VMEM((_BPW, 8), jnp.float32) for _ in range(_NSC)]
        + [pltpu.VMEM((8 * _NSC, _BPW), jnp.float32),
           pltpu.SemaphoreType.DMA]
    )

    def body(xt_hbm, *refs):
        tabs = refs[:_NSC]
        out = refs[_NSC]
        xcol = refs[_NSC + 1]
        idx_v = refs[_NSC + 2]
        rows = refs[_NSC + 3:_NSC + 3 + _NSC]
        st = refs[_NSC + 3 + _NSC]
        sem = refs[_NSC + 4 + _NSC]

        wid = lax.axis_index("s") * _NC + lax.axis_index("c")
        base = wid * _BPW
        lanes = lax.iota(jnp.int32, _L)
        for t in range(_NSC):
            pltpu.sync_copy(xt_hbm.at[t, pl.ds(base, _BPW)], xcol)
            for c in range(_BPW // _L):
                vals = xcol[pl.ds(c * _L, _L)].astype(jnp.int32)
                j, o = divmod(c * _L, _CHUNK)
                idx_v[t, j, pl.ds(o, _L)] = vals
        copies = []
        for t in range(_NSC):
            for j in range(_BPW // _CHUNK):
                copies.append(pltpu.async_copy(
                    tabs[t].at[idx_v.at[t, j]],
                    rows[t].at[pl.ds(j * _CHUNK, _CHUNK), :], sem))
        for cp in copies:
            cp.wait()
        for t in range(_NSC):
            for d in range(8):
                dv = jnp.full((_L,), d, jnp.int32)
                for g in range(_BPW // _L):
                    vals = plsc.load_gather(rows[t], [g * _L + lanes, dv])
                    st[t * 8 + d, pl.ds(g * _L, _L)] = vals
        pltpu.sync_copy(st, out.at[:, pl.ds(base, _BPW)])

    fn = pl.kernel(
        body, out_type=out_type, mesh=mesh, scratch_types=scratch,
        compiler_params=pltpu.CompilerParams(
            use_tc_tiling_on_sc=False, needs_layout_passes=False))
    return fn(xt, *tables)


_OFFS2 = []
_o2 = 0
for _d2 in _DIMS:
    _OFFS2.append(_o2)
    _o2 += _d2
_TIME_ROW = _o2


def _tc_pre(xt, small_t, W1t, b1):
    blk = 2048
    grid = (_B // blk,)

    def body(*refs):
        x_ref = refs[0]
        tt = refs[1:7]
        w1t, b1r = refs[7:9]
        out = refs[9]
        xb = x_ref[...]
        w1v = w1t[...]
        h = b1r[...] + w1v[:, _TIME_ROW:_TIME_ROW + 1] * xb[9:10, :]
        for k in range(6):
            t = _NSC + k
            v = _VOCABS[t] + 1
            projT = jnp.dot(w1v[:, _OFFS2[t]:_OFFS2[t] + _DIMS[t]], tt[k][...],
                            preferred_element_type=jnp.float32)
            ids = lax.broadcasted_iota(jnp.int32, (v, blk), 0)
            onehotT = (ids == xb[t:t + 1, :].astype(jnp.int32)).astype(jnp.float32)
            h = h + jnp.dot(projT, onehotT, preferred_element_type=jnp.float32)
        out[...] = h

    in_specs = (
        [pl.BlockSpec((10, blk), lambda i: (0, i))]
        + [pl.BlockSpec(t.shape, lambda i: (0, 0)) for t in small_t]
        + [pl.BlockSpec((16, 146), lambda i: (0, 0)),
           pl.BlockSpec((16, 1), lambda i: (0, 0))]
    )
    return pl.pallas_call(
        body,
        grid=grid,
        in_specs=in_specs,
        out_specs=pl.BlockSpec((16, blk), lambda i: (0, i)),
        out_shape=jax.ShapeDtypeStruct((16, _B), jnp.float32),
    )(xt, *small_t, W1t, b1)


def _tc_post(hpre, gt, W1t, W2t, b2, W3t, b3):
    blk = 2048
    grid = (_B // blk,)

    def body(h_ref, g_ref, w1t, w2t, b2r, w3t, b3r, out):
        gb = g_ref[...]
        w1v = w1t[...]
        h = h_ref[...] + jnp.dot(w1v[:, 0:24], gb,
                                 preferred_element_type=jnp.float32)
        h = jnp.maximum(h, 0.0)
        h = jnp.maximum(jnp.dot(w2t[...], h, preferred_element_type=jnp.float32)
                        + b2r[...], 0.0)
        o1 = jnp.dot(w3t[...], h, preferred_element_type=jnp.float32) + b3r[...]
        out[...] = jnp.broadcast_to(jax.nn.sigmoid(o1), (8, blk))

    in_specs = [pl.BlockSpec((16, blk), lambda i: (0, i)),
                pl.BlockSpec((8 * _NSC, blk), lambda i: (0, i)),
                pl.BlockSpec((16, 146), lambda i: (0, 0)),
                pl.BlockSpec((4, 16), lambda i: (0, 0)),
                pl.BlockSpec((4, 1), lambda i: (0, 0)),
                pl.BlockSpec((1, 4), lambda i: (0, 0)),
                pl.BlockSpec((1, 1), lambda i: (0, 0))]
    return pl.pallas_call(
        body,
        grid=grid,
        in_specs=in_specs,
        out_specs=pl.BlockSpec((8, blk), lambda i: (0, i)),
        out_shape=jax.ShapeDtypeStruct((8, _B), jnp.float32),
    )(hpre, gt, W1t, W2t, b2, W3t, b3)


def kernel(x, user_emb, product_emb, model_emb, gender_emb, age_emb,
           residence_emb, color_emb, size_emb, material_emb,
           W1, b1, W2, b2, W3, b3):
    xt = x.T
    utail = jnp.pad(user_emb[488 * _KT * 128:], ((0, 5 * 128 - 577), (0, 0)))
    ptail = jnp.pad(product_emb[48 * _KT * 128:], ((0, 14 * 128 - 1697), (0, 0)))
    mpad = jnp.pad(model_emb, ((0, 8 * 128 - 1001), (0, 0)))
    uo, po, mo = _detile(user_emb.T, utail.T, product_emb.T, ptail.T, mpad.T)
    big = tuple(o.reshape(n * 8 * 16, 8) for o, n in zip((uo, po, mo), _NTILES))
    gt = _sc_gather(xt, *big)
    small_t = (gender_emb.T, age_emb.T, residence_emb.T, color_emb.T,
               size_emb.T, material_emb.T)
    hpre = _tc_pre(xt, small_t, W1.T, b1.reshape(16, 1))
    out = _tc_post(hpre, gt, W1.T, W2.T, b2.reshape(4, 1), W3.T,
                   b3.reshape(1, 1))
    return out[0]
